# baseline (speedup 1.0000x reference)
import functools

import jax
import jax.numpy as jnp
from jax.experimental import pallas as pl
from jax.experimental.pallas import tpu as pltpu

_N = 10000
_E = 320000
_F = 128
_D = 128
_A = 8192
_STEPS = 8
_C = 10


def _in_tf_body(x_ref, w_ref, b_ref, o_ref):
    v = jnp.dot(x_ref[...], w_ref[...], preferred_element_type=jnp.float32)
    v = v + b_ref[...]
    o_ref[...] = jax.nn.leaky_relu(v, negative_slope=0.01)


def _input_transform(x, W_in, b_in):
    return pl.pallas_call(
        _in_tf_body,
        out_shape=jax.ShapeDtypeStruct((_N, _D), jnp.float32),
    )(x, W_in, b_in.reshape(1, _D))


def _agent_body(st_ref, na_ref, wa_ref, ba_ref, wn_ref, bn_ref, st_o, upd_o):
    cat = jnp.concatenate([st_ref[...], na_ref[...]], axis=-1)
    ns = jnp.dot(cat, wa_ref[...], preferred_element_type=jnp.float32) + ba_ref[...]
    ns = jax.nn.leaky_relu(ns, negative_slope=0.01)
    st_o[...] = ns
    upd_o[...] = jnp.dot(ns, wn_ref[...], preferred_element_type=jnp.float32) + bn_ref[...]


def _agent_step(state, node_at, W_agent, b_agent, W_node, b_node):
    return pl.pallas_call(
        _agent_body,
        out_shape=(
            jax.ShapeDtypeStruct((_A, _D), jnp.float32),
            jax.ShapeDtypeStruct((_A, _D), jnp.float32),
        ),
    )(state, node_at, W_agent, b_agent.reshape(1, _D), W_node, b_node.reshape(1, _D))


def _msg_body(h_ref, u_ref, wm_ref, bm_ref, h2_o, hm_o):
    h2 = h_ref[...] + u_ref[...]
    h2_o[...] = h2
    v = jnp.dot(h2, wm_ref[...], preferred_element_type=jnp.float32) + bm_ref[...]
    hm_o[...] = jax.nn.leaky_relu(v, negative_slope=0.2)


def _msg_step(h, U, W_msg, b_msg):
    return pl.pallas_call(
        _msg_body,
        out_shape=(
            jax.ShapeDtypeStruct((_N, _D), jnp.float32),
            jax.ShapeDtypeStruct((_N, _D), jnp.float32),
        ),
    )(h, U, W_msg, b_msg.reshape(1, _D))


def _score_body(h2_ref, agg_ref, aa_ref, h3_o, s_o):
    h3 = jax.nn.leaky_relu(h2_ref[...] + agg_ref[...], negative_slope=0.01)
    h3_o[...] = h3
    s_o[...] = jnp.dot(h3, aa_ref[...], preferred_element_type=jnp.float32)


def _score_step(h2, agg, a_src, a_dst):
    aa = jnp.stack([a_src, a_dst], axis=-1)
    return pl.pallas_call(
        _score_body,
        out_shape=(
            jax.ShapeDtypeStruct((_N, _D), jnp.float32),
            jax.ShapeDtypeStruct((_N, 2), jnp.float32),
        ),
    )(h2, agg, aa)


def kernel(x, edge_index, agent_pos, W_in, b_in, agent_emb, W_agent, b_agent,
           W_node, b_node, a_src, a_dst, W_msg, b_msg, W_ro, b_ro):
    src = edge_index[0]
    dst = edge_index[1]
    h = _input_transform(x, W_in, b_in)
    agent_state = agent_emb
    pos = agent_pos
    for _ in range(_STEPS):
        node_at = jnp.take(h, pos, axis=0)
        agent_state, upd = _agent_step(agent_state, node_at, W_agent, b_agent,
                                       W_node, b_node)
        U = jax.ops.segment_sum(upd, pos, num_segments=_N)
        h2, hm = _msg_step(h, U, W_msg, b_msg)
        agg = jax.ops.segment_sum(jnp.take(hm, src, axis=0), dst, num_segments=_N)
        h, s = _score_step(h2, agg, a_src, a_dst)
        score = jax.nn.leaky_relu(s[src, 0] + s[dst, 1], negative_slope=0.2)
        seg_max = jax.ops.segment_max(score, src, num_segments=_N)
        is_best = score >= (jnp.take(seg_max, src) - 1e-6)
        best_dst = jax.ops.segment_max(jnp.where(is_best, dst, -1), src,
                                       num_segments=_N)
        cand = jnp.take(best_dst, pos)
        pos = jnp.where(cand >= 0, cand, pos)
    node_pool = jnp.mean(h, axis=0)
    agent_pool = jnp.mean(agent_state, axis=0)
    out = (node_pool + agent_pool) @ W_ro + b_ro
    return out[None, :]


# speedup vs baseline: 1.0970x; 1.0970x over previous
import jax
import jax.numpy as jnp
from jax import lax
from jax.experimental import pallas as pl
from jax.experimental.pallas import tpu as pltpu
from jax.experimental.pallas import tpu_sc as plsc

_N = 10000
_E = 320000
_F = 128
_D = 128
_A = 8192
_STEPS = 8
_C = 10

_NC = 2
_NS = 16
_NW = _NC * _NS

_RPW = 312
_HALF = _RPW * _NS
_TRASH = 5008
_ACC_ROWS = 5016
_SB = 4096
_CK = 128
_EPAD = _E + _SB + 8


def _in_tf_body(x_ref, w_ref, b_ref, o_ref):
    v = jnp.dot(x_ref[...], w_ref[...], preferred_element_type=jnp.float32)
    v = v + b_ref[...]
    o_ref[...] = jax.nn.leaky_relu(v, negative_slope=0.01)


def _input_transform(x, W_in, b_in):
    return pl.pallas_call(
        _in_tf_body,
        out_shape=jax.ShapeDtypeStruct((_N, _D), jnp.float32),
    )(x, W_in, b_in.reshape(1, _D))


def _agent_body(st_ref, na_ref, wa_ref, ba_ref, wn_ref, bn_ref, st_o, upd_o):
    cat = jnp.concatenate([st_ref[...], na_ref[...]], axis=-1)
    ns = jnp.dot(cat, wa_ref[...], preferred_element_type=jnp.float32) + ba_ref[...]
    ns = jax.nn.leaky_relu(ns, negative_slope=0.01)
    st_o[...] = ns
    upd_o[...] = jnp.dot(ns, wn_ref[...], preferred_element_type=jnp.float32) + bn_ref[...]


def _agent_step(state, node_at, W_agent, b_agent, W_node, b_node):
    return pl.pallas_call(
        _agent_body,
        out_shape=(
            jax.ShapeDtypeStruct((_A, _D), jnp.float32),
            jax.ShapeDtypeStruct((_A, _D), jnp.float32),
        ),
    )(state, node_at, W_agent, b_agent.reshape(1, _D), W_node, b_node.reshape(1, _D))


def _msg_body(h2_ref, wm_ref, bm_ref, hm_o):
    v = jnp.dot(h2_ref[...], wm_ref[...], preferred_element_type=jnp.float32)
    hm_o[...] = jax.nn.leaky_relu(v + bm_ref[...], negative_slope=0.2)


def _msg_step(h2, W_msg, b_msg):
    return pl.pallas_call(
        _msg_body,
        out_shape=jax.ShapeDtypeStruct((_N, _D), jnp.float32),
    )(h2, W_msg, b_msg.reshape(1, _D))


def _score_body(h2_ref, agg_ref, aa_ref, h3_o, s_o):
    h3 = jax.nn.leaky_relu(h2_ref[...] + agg_ref[...], negative_slope=0.01)
    h3_o[...] = h3
    s_o[...] = jnp.dot(h3, aa_ref[...], preferred_element_type=jnp.float32)


def _score_step(h2, agg, a_src, a_dst):
    aa = jnp.stack([a_src, a_dst], axis=-1)
    return pl.pallas_call(
        _score_body,
        out_shape=(
            jax.ShapeDtypeStruct((_N, _D), jnp.float32),
            jax.ShapeDtypeStruct((_N, 2), jnp.float32),
        ),
    )(h2, agg, aa)


_MESH = plsc.VectorSubcoreMesh(core_axis_name="c", subcore_axis_name="s")


def _gather_rows_body(h_hbm, pos_hbm, out_hbm, idx, rows, sem):
    c = lax.axis_index("c")
    s = lax.axis_index("s")
    base = (c * _NS + s) * (_A // _NW)
    k = 128
    for t in range(2):
        off = base + t * k
        pltpu.sync_copy(pos_hbm.at[pl.ds(off, k)], idx)
        pltpu.async_copy(h_hbm.at[idx], rows, sem).wait()
        pltpu.sync_copy(rows, out_hbm.at[pl.ds(off, k)])


_gather_agent_rows = pl.kernel(
    _gather_rows_body,
    out_type=jax.ShapeDtypeStruct((_A, _D), jnp.float32),
    mesh=_MESH,
    scratch_types=[
        pltpu.VMEM((128,), jnp.int32),
        pltpu.VMEM((128, _D), jnp.float32),
        pltpu.SemaphoreType.DMA,
    ],
)


def _seg_sum_csr_body(hm_hbm, srcp_hbm, dstl_hbm, rp_hbm, zer_hbm, out_hbm,
                      rpb, sall, dall, didx, rows, acc, sem):
    c = lax.axis_index("c")
    s = lax.axis_index("s")
    w = c * _NS + s
    is_last = (c == _NC - 1) & (s == _NS - 1)

    pltpu.sync_copy(rp_hbm.at[pl.ds(w * _RPW, 344)], rpb)
    e_lo = rpb[pl.ds(0, 16)][0]
    e_hi = jnp.where(is_last, rpb[pl.ds(_RPW + 16, 16)][0],
                     rpb[pl.ds(_RPW, 16)][0])
    c0 = (e_lo // 8) * 8

    pltpu.sync_copy(zer_hbm.at[pl.ds(0, _RPW)], acc.at[pl.ds(s * _RPW, _RPW)])

    @pl.when(is_last)
    def _():
        pltpu.sync_copy(zer_hbm.at[pl.ds(_RPW, 16)], acc.at[pl.ds(_HALF, 16)])

    nch = (e_hi - c0 + (_CK - 1)) // _CK
    spc = _SB // _CK

    def ch_body(i, carry):
        @pl.when(i % spc == 0)
        def _():
            off0 = c0 + (i // spc) * _SB
            pltpu.sync_copy(srcp_hbm.at[pl.ds(off0, _SB)], sall)
            pltpu.sync_copy(dstl_hbm.at[pl.ds(off0, _SB)], dall)

        loc = (i % spc) * _CK
        gbase = c0 + i * _CK
        for v in range(_CK // 16):
            gid = gbase + v * 16 + lax.iota(jnp.int32, 16)
            dv = dall[pl.ds(loc + v * 16, 16)]
            valid = (gid >= e_lo) & (gid < e_hi)
            didx[pl.ds(v * 16, 16)] = jnp.where(valid, dv, _TRASH)
        pltpu.async_copy(hm_hbm.at[sall.at[pl.ds(loc, _CK)]], rows, sem).wait()
        pltpu.sync_copy(rows, acc.at[didx], add=True)
        return carry

    lax.fori_loop(0, nch, ch_body, 0)

    pltpu.sync_copy(acc.at[pl.ds(s * _RPW, _RPW)],
                    out_hbm.at[pl.ds(w * _RPW, _RPW)])

    @pl.when(is_last)
    def _():
        pltpu.sync_copy(acc.at[pl.ds(_HALF, 16)],
                        out_hbm.at[pl.ds(_NS * _RPW * _NC, 16)])


_seg_sum_csr = pl.kernel(
    _seg_sum_csr_body,
    out_type=jax.ShapeDtypeStruct((_N, _D), jnp.float32),
    mesh=_MESH,
    scratch_types=[
        pltpu.VMEM((344,), jnp.int32),
        pltpu.VMEM((_SB,), jnp.int32),
        pltpu.VMEM((_SB,), jnp.int32),
        pltpu.VMEM((_CK,), jnp.int32),
        pltpu.VMEM((_CK, _D), jnp.float32),
        pltpu.VMEM_SHARED((_ACC_ROWS, _D), jnp.float32),
        pltpu.SemaphoreType.DMA,
    ],
)


def _build_dst_csr(dst):
    order = jnp.argsort(dst, stable=True)
    dstp = jnp.take(dst, order)
    row_ptr = jnp.searchsorted(dstp, jnp.arange(_N + 1, dtype=dst.dtype),
                               side="left").astype(jnp.int32)
    rp_full = jnp.full((_NW * _RPW + 344,), _E, jnp.int32)
    rp_full = rp_full.at[: _N + 1].set(row_ptr)
    dstl = dstp - jnp.where(dstp >= _HALF, _HALF, 0)
    dstl_pad = jnp.full((_EPAD,), _TRASH, jnp.int32).at[:_E].set(dstl)
    return order, rp_full, dstl_pad


def kernel(x, edge_index, agent_pos, W_in, b_in, agent_emb, W_agent, b_agent,
           W_node, b_node, a_src, a_dst, W_msg, b_msg, W_ro, b_ro):
    src = edge_index[0]
    dst = edge_index[1]
    zeros_nd = jnp.zeros((_N, _D), jnp.float32)

    order, rp_full, dstl_pad = _build_dst_csr(dst)
    srcp = jnp.take(src, order)
    srcp_pad = jnp.zeros((_EPAD,), jnp.int32).at[:_E].set(srcp.astype(jnp.int32))

    h = _input_transform(x, W_in, b_in)
    agent_state = agent_emb
    pos = agent_pos
    for _ in range(_STEPS):
        node_at = _gather_agent_rows(h, pos)
        agent_state, upd = _agent_step(agent_state, node_at, W_agent, b_agent,
                                       W_node, b_node)
        h2 = h.at[pos].add(upd)
        hm = _msg_step(h2, W_msg, b_msg)
        agg = _seg_sum_csr(hm, srcp_pad, dstl_pad, rp_full, zeros_nd)
        h, sc = _score_step(h2, agg, a_src, a_dst)
        score = jax.nn.leaky_relu(sc[src, 0] + sc[dst, 1], negative_slope=0.2)
        seg_max = jax.ops.segment_max(score, src, num_segments=_N)
        is_best = score >= (jnp.take(seg_max, src) - 1e-6)
        best_dst = jax.ops.segment_max(jnp.where(is_best, dst, -1), src,
                                       num_segments=_N)
        cand = jnp.take(best_dst, pos)
        pos = jnp.where(cand >= 0, cand, pos)
    node_pool = jnp.mean(h, axis=0)
    agent_pool = jnp.mean(agent_state, axis=0)
    out = (node_pool + agent_pool) @ W_ro + b_ro
    return out[None, :]


# speedup vs baseline: 20.1238x; 18.3439x over previous
import jax
import jax.numpy as jnp
from jax import lax
from jax.experimental import pallas as pl
from jax.experimental.pallas import tpu as pltpu
from jax.experimental.pallas import tpu_sc as plsc

_N = 10000
_E = 320000
_F = 128
_D = 128
_A = 8192
_STEPS = 8
_C = 10

_NC = 2
_NS = 16
_NW = _NC * _NS

_RPW = 312
_HALF = _RPW * _NS
_TRASH = 5008
_ACC_ROWS = 5016
_SB = 4096
_CK = 128
_EPAD = _E + _SB + 8


def _in_tf_body(x_ref, w_ref, b_ref, o_ref):
    v = jnp.dot(x_ref[...], w_ref[...], preferred_element_type=jnp.float32)
    v = v + b_ref[...]
    o_ref[...] = jax.nn.leaky_relu(v, negative_slope=0.01)


def _input_transform(x, W_in, b_in):
    return pl.pallas_call(
        _in_tf_body,
        out_shape=jax.ShapeDtypeStruct((_N, _D), jnp.float32),
    )(x, W_in, b_in.reshape(1, _D))


def _agent_body(st_ref, na_ref, wa_ref, ba_ref, wn_ref, bn_ref, st_o, upd_o):
    cat = jnp.concatenate([st_ref[...], na_ref[...]], axis=-1)
    ns = jnp.dot(cat, wa_ref[...], preferred_element_type=jnp.float32) + ba_ref[...]
    ns = jax.nn.leaky_relu(ns, negative_slope=0.01)
    st_o[...] = ns
    upd_o[...] = jnp.dot(ns, wn_ref[...], preferred_element_type=jnp.float32) + bn_ref[...]


def _agent_step(state, node_at, W_agent, b_agent, W_node, b_node):
    return pl.pallas_call(
        _agent_body,
        out_shape=(
            jax.ShapeDtypeStruct((_A, _D), jnp.float32),
            jax.ShapeDtypeStruct((_A, _D), jnp.float32),
        ),
    )(state, node_at, W_agent, b_agent.reshape(1, _D), W_node, b_node.reshape(1, _D))


def _msg_body(h2_ref, wm_ref, bm_ref, hm_o):
    v = jnp.dot(h2_ref[...], wm_ref[...], preferred_element_type=jnp.float32)
    hm_o[...] = jax.nn.leaky_relu(v + bm_ref[...], negative_slope=0.2)


def _msg_step(h2, W_msg, b_msg):
    return pl.pallas_call(
        _msg_body,
        out_shape=jax.ShapeDtypeStruct((_N, _D), jnp.float32),
    )(h2, W_msg, b_msg.reshape(1, _D))


def _score_body(h2_ref, agg_ref, aa_ref, h3_o, s_o):
    h3 = jax.nn.leaky_relu(h2_ref[...] + agg_ref[...], negative_slope=0.01)
    h3_o[...] = h3
    s_o[...] = jnp.dot(h3, aa_ref[...], preferred_element_type=jnp.float32)


def _score_step(h2, agg, a_src, a_dst):
    aa = jnp.stack([a_src, a_dst], axis=-1)
    return pl.pallas_call(
        _score_body,
        out_shape=(
            jax.ShapeDtypeStruct((_N, _D), jnp.float32),
            jax.ShapeDtypeStruct((_N, 2), jnp.float32),
        ),
    )(h2, agg, aa)


_MESH = plsc.VectorSubcoreMesh(core_axis_name="c", subcore_axis_name="s")


def _gather_pos_rows_body(h_hbm, pos_hbm, bd_hbm, pos_out, na_out,
                          bdv, idx, npos, rows, sem):
    c = lax.axis_index("c")
    s = lax.axis_index("s")
    base = (c * _NS + s) * (_A // _NW)
    pltpu.sync_copy(bd_hbm, bdv)
    k = 128
    for t in range(2):
        off = base + t * k
        pltpu.sync_copy(pos_hbm.at[pl.ds(off, k)], idx)
        for v in range(k // 16):
            pv = idx[pl.ds(v * 16, 16)]
            cand = plsc.load_gather(bdv, [pv])
            npos[pl.ds(v * 16, 16)] = jnp.where(cand >= 0, cand, pv)
        pltpu.async_copy(h_hbm.at[npos], rows, sem).wait()
        pltpu.sync_copy(npos, pos_out.at[pl.ds(off, k)])
        pltpu.sync_copy(rows, na_out.at[pl.ds(off, k)])


_gather_pos_rows = pl.kernel(
    _gather_pos_rows_body,
    out_type=(
        jax.ShapeDtypeStruct((_A,), jnp.int32),
        jax.ShapeDtypeStruct((_A, _D), jnp.float32),
    ),
    mesh=_MESH,
    scratch_types=[
        pltpu.VMEM((_N,), jnp.int32),
        pltpu.VMEM((128,), jnp.int32),
        pltpu.VMEM((128,), jnp.int32),
        pltpu.VMEM((128, _D), jnp.float32),
        pltpu.SemaphoreType.DMA,
    ],
    compiler_params=pltpu.CompilerParams(needs_layout_passes=False),
)


_NLOC = 352
_LTRASH = 336


def _transition_body(s1_hbm, s2_hbm, srcs_hbm, dsts_hbm, rp2_hbm, out_hbm,
                     rpb, s1v, s2v, sall, dall, smax, best):
    c = lax.axis_index("c")
    s = lax.axis_index("s")
    w = c * _NS + s
    is_last = (c == _NC - 1) & (s == _NS - 1)
    rowbase = w * _RPW

    pltpu.sync_copy(rp2_hbm.at[pl.ds(w * _RPW, 344)], rpb)
    e_lo = rpb[pl.ds(0, 16)][0]
    e_hi = jnp.where(is_last, rpb[pl.ds(_RPW + 16, 16)][0],
                     rpb[pl.ds(_RPW, 16)][0])
    c0 = (e_lo // 8) * 8

    pltpu.sync_copy(s1_hbm, s1v)
    pltpu.sync_copy(s2_hbm, s2v)
    for t in range(_NLOC // 16):
        smax[pl.ds(t * 16, 16)] = jnp.full((16,), -jnp.inf, jnp.float32)
        best[pl.ds(t * 16, 16)] = jnp.full((16,), -1, jnp.int32)

    nch = (e_hi - c0 + (_CK - 1)) // _CK
    spc = _SB // _CK
    iota = lax.iota(jnp.int32, 16)
    nxt = jnp.minimum(iota + 1, 15)

    def run_pass(second):
        def ch_body(i, carry):
            @pl.when(i % spc == 0)
            def _():
                off0 = c0 + (i // spc) * _SB
                pltpu.sync_copy(srcs_hbm.at[pl.ds(off0, _SB)], sall)
                pltpu.sync_copy(dsts_hbm.at[pl.ds(off0, _SB)], dall)

            loc = (i % spc) * _CK
            gbase = c0 + i * _CK
            for v in range(_CK // 16):
                sv = sall[pl.ds(loc + v * 16, 16)]
                dv = dall[pl.ds(loc + v * 16, 16)]
                gid = gbase + v * 16 + iota
                valid = (gid >= e_lo) & (gid < e_hi)
                v1 = plsc.load_gather(s1v, [sv])
                v2 = plsc.load_gather(s2v, [dv])
                sc = v1 + v2
                sc = jnp.where(sc >= 0, sc, 0.2 * sc)
                lv = jnp.where(valid, sv - rowbase, _LTRASH)
                knext = jnp.take(sv, nxt)
                last = (sv != knext) | (iota == 15)
                if not second:
                    val = jnp.where(valid, sc, -jnp.inf)
                    key = sv
                    for sh in (1, 2, 4, 8):
                        pidx = jnp.maximum(iota - sh, 0)
                        kp = jnp.take(key, pidx)
                        vp = jnp.take(val, pidx)
                        val = jnp.where(kp == key, jnp.maximum(val, vp), val)
                    cur = plsc.load_gather(smax, [lv])
                    plsc.store_scatter(smax, [lv], jnp.maximum(cur, val),
                                       mask=last)
                else:
                    th = plsc.load_gather(smax, [lv]) - 1e-6
                    cand = jnp.where(valid & (sc >= th), dv, -1)
                    key = sv
                    for sh in (1, 2, 4, 8):
                        pidx = jnp.maximum(iota - sh, 0)
                        kp = jnp.take(key, pidx)
                        cp = jnp.take(cand, pidx)
                        cand = jnp.where(kp == key, jnp.maximum(cand, cp), cand)
                    cur = plsc.load_gather(best, [lv])
                    plsc.store_scatter(best, [lv], jnp.maximum(cur, cand),
                                       mask=last)
            return carry

        lax.fori_loop(0, nch, ch_body, 0)

    run_pass(False)
    run_pass(True)

    pltpu.sync_copy(best.at[pl.ds(0, _RPW)], out_hbm.at[pl.ds(w * _RPW, _RPW)])

    @pl.when(is_last)
    def _():
        pltpu.sync_copy(best.at[pl.ds(_RPW, 16)],
                        out_hbm.at[pl.ds(_NW * _RPW, 16)])


_transition = pl.kernel(
    _transition_body,
    out_type=jax.ShapeDtypeStruct((_N,), jnp.int32),
    mesh=_MESH,
    scratch_types=[
        pltpu.VMEM((344,), jnp.int32),
        pltpu.VMEM((_N + 48,), jnp.float32),
        pltpu.VMEM((_N + 48,), jnp.float32),
        pltpu.VMEM((_SB,), jnp.int32),
        pltpu.VMEM((_SB,), jnp.int32),
        pltpu.VMEM((_NLOC,), jnp.float32),
        pltpu.VMEM((_NLOC,), jnp.int32),
    ],
    compiler_params=pltpu.CompilerParams(needs_layout_passes=False),
)


def _seg_sum_csr_body(hm_hbm, srcp_hbm, dstl_hbm, rp_hbm, zer_hbm, out_hbm,
                      rpb, sall, dall, didx, rows, acc, sem):
    c = lax.axis_index("c")
    s = lax.axis_index("s")
    w = c * _NS + s
    is_last = (c == _NC - 1) & (s == _NS - 1)

    pltpu.sync_copy(rp_hbm.at[pl.ds(w * _RPW, 344)], rpb)
    e_lo = rpb[pl.ds(0, 16)][0]
    e_hi = jnp.where(is_last, rpb[pl.ds(_RPW + 16, 16)][0],
                     rpb[pl.ds(_RPW, 16)][0])
    c0 = (e_lo // 8) * 8

    pltpu.sync_copy(zer_hbm.at[pl.ds(0, _RPW)], acc.at[pl.ds(s * _RPW, _RPW)])

    @pl.when(is_last)
    def _():
        pltpu.sync_copy(zer_hbm.at[pl.ds(_RPW, 16)], acc.at[pl.ds(_HALF, 16)])

    nch = (e_hi - c0 + (_CK - 1)) // _CK
    spc = _SB // _CK

    def ch_body(i, carry):
        @pl.when(i % spc == 0)
        def _():
            off0 = c0 + (i // spc) * _SB
            pltpu.sync_copy(srcp_hbm.at[pl.ds(off0, _SB)], sall)
            pltpu.sync_copy(dstl_hbm.at[pl.ds(off0, _SB)], dall)

        loc = (i % spc) * _CK
        gbase = c0 + i * _CK
        for v in range(_CK // 16):
            gid = gbase + v * 16 + lax.iota(jnp.int32, 16)
            dv = dall[pl.ds(loc + v * 16, 16)]
            valid = (gid >= e_lo) & (gid < e_hi)
            didx[pl.ds(v * 16, 16)] = jnp.where(valid, dv, _TRASH)
        pltpu.async_copy(hm_hbm.at[sall.at[pl.ds(loc, _CK)]], rows, sem).wait()
        pltpu.sync_copy(rows, acc.at[didx], add=True)
        return carry

    lax.fori_loop(0, nch, ch_body, 0)

    pltpu.sync_copy(acc.at[pl.ds(s * _RPW, _RPW)],
                    out_hbm.at[pl.ds(w * _RPW, _RPW)])

    @pl.when(is_last)
    def _():
        pltpu.sync_copy(acc.at[pl.ds(_HALF, 16)],
                        out_hbm.at[pl.ds(_NS * _RPW * _NC, 16)])


_seg_sum_csr = pl.kernel(
    _seg_sum_csr_body,
    out_type=jax.ShapeDtypeStruct((_N, _D), jnp.float32),
    mesh=_MESH,
    scratch_types=[
        pltpu.VMEM((344,), jnp.int32),
        pltpu.VMEM((_SB,), jnp.int32),
        pltpu.VMEM((_SB,), jnp.int32),
        pltpu.VMEM((_CK,), jnp.int32),
        pltpu.VMEM((_CK, _D), jnp.float32),
        pltpu.VMEM_SHARED((_ACC_ROWS, _D), jnp.float32),
        pltpu.SemaphoreType.DMA,
    ],
)


def _build_csr(keys):
    order = jnp.argsort(keys, stable=True)
    sorted_keys = jnp.take(keys, order)
    row_ptr = jnp.searchsorted(sorted_keys, jnp.arange(_N + 1, dtype=keys.dtype),
                               side="left").astype(jnp.int32)
    rp_full = jnp.full((_NW * _RPW + 344,), _E, jnp.int32)
    rp_full = rp_full.at[: _N + 1].set(row_ptr)
    return order, sorted_keys, rp_full


def _pad_edges(arr, fill):
    return jnp.full((_EPAD,), fill, jnp.int32).at[:_E].set(arr.astype(jnp.int32))


def kernel(x, edge_index, agent_pos, W_in, b_in, agent_emb, W_agent, b_agent,
           W_node, b_node, a_src, a_dst, W_msg, b_msg, W_ro, b_ro):
    src = edge_index[0]
    dst = edge_index[1]
    zeros_nd = jnp.zeros((_N, _D), jnp.float32)

    order_d, dstp, rp_d = _build_csr(dst)
    dstl = dstp - jnp.where(dstp >= _HALF, _HALF, 0)
    dstl_pad = _pad_edges(dstl, _TRASH)
    srcp_pad = _pad_edges(jnp.take(src, order_d), 0)

    order_s, srcs, rp_s = _build_csr(src)
    srcs_pad = _pad_edges(srcs, _N)
    dsts_pad = _pad_edges(jnp.take(dst, order_s), 0)

    h = _input_transform(x, W_in, b_in)
    agent_state = agent_emb
    pos = agent_pos
    best_dst = jnp.full((_N,), -1, jnp.int32)
    for _ in range(_STEPS):
        pos, node_at = _gather_pos_rows(h, pos, best_dst)
        agent_state, upd = _agent_step(agent_state, node_at, W_agent, b_agent,
                                       W_node, b_node)
        h2 = h.at[pos].add(upd)
        hm = _msg_step(h2, W_msg, b_msg)
        agg = _seg_sum_csr(hm, srcp_pad, dstl_pad, rp_d, zeros_nd)
        h, sc = _score_step(h2, agg, a_src, a_dst)
        s1 = jnp.zeros((_N + 48,), jnp.float32).at[:_N].set(sc[:, 0])
        s2 = jnp.zeros((_N + 48,), jnp.float32).at[:_N].set(sc[:, 1])
        best_dst = _transition(s1, s2, srcs_pad, dsts_pad, rp_s)
    node_pool = jnp.mean(h, axis=0)
    agent_pool = jnp.mean(agent_state, axis=0)
    out = (node_pool + agent_pool) @ W_ro + b_ro
    return out[None, :]
